# baseline (speedup 1.0000x reference)
import jax
import jax.numpy as jnp
from jax.experimental import pallas as pl
from jax.experimental.pallas import tpu as pltpu
from jax.experimental.pallas import tpu_sc as plsc

_CHUNK = 64
_GATHER_WINDOW = 128
_HEAD_BLOCKS = 8


def _sc_gather(table, idx):
    n = idx.shape[0]
    e = table.shape[1]
    idx2 = idx.reshape(1, n)
    mesh = plsc.VectorSubcoreMesh(core_axis_name="core",
                                  subcore_axis_name="subcore")

    @pl.kernel(out_type=jax.ShapeDtypeStruct((n, e), table.dtype), mesh=mesh)
    def _gather_kernel(tab_hbm, i_hbm, o_hbm):
        def body(i_vmem, o_vmem):
            pltpu.sync_copy(tab_hbm.at[i_vmem.at[0]], o_vmem)

        pltpu.emit_pipeline(
            body,
            grid=(n // _GATHER_WINDOW,),
            in_specs=[pl.BlockSpec((1, _GATHER_WINDOW),
                                   index_map=lambda i: (0, i))],
            out_specs=[pl.BlockSpec((_GATHER_WINDOW, e),
                                    index_map=lambda i: (i, 0))],
            core_axis_name=("core", "subcore"),
            dimension_semantics=(pltpu.PARALLEL,),
        )(i_hbm, o_hbm)

    return _gather_kernel(table, idx2)


def _dot(a, b):
    return jax.lax.dot_general(a.astype(jnp.bfloat16), b.astype(jnp.bfloat16),
                               (((1,), (0,)), ((), ())),
                               preferred_element_type=jnp.float32)


def _sig(x):
    return 0.5 * jnp.tanh(0.5 * x) + 0.5


def _lstm_cell(gates, cv, h):
    ig = gates[:, :h]
    fg = gates[:, h:2 * h]
    gg = gates[:, 2 * h:3 * h]
    og = gates[:, 3 * h:]
    cc = _sig(fg) * cv + _sig(ig) * jnp.tanh(gg)
    hh = _sig(og) * jnp.tanh(cc)
    return hh, cc


def _lstm_body(embeds_ref, wih0_ref, whh0_ref, bias0_ref,
               wih1_ref, whh1_ref, bias1_ref,
               ys1_ref, hn_ref, cn_ref, x_scr):
    nlayers, b, h = hn_ref.shape
    sb = embeds_ref.shape[0]
    seq = sb // b
    nchunks = seq // _CHUNK

    def chunk(k, carry):
        x_scr[...] = (_dot(embeds_ref[pl.ds(k * _CHUNK * b, _CHUNK * b), :],
                           wih0_ref[...]) + bias0_ref[...]).astype(x_scr.dtype)

        def step(i, hc):
            h0, c0, h1, c1 = hc
            j = k * _CHUNK + i
            gates1 = _dot(h0, wih1_ref[...]) + _dot(h1, whh1_ref[...]) + \
                bias1_ref[...]
            h1n, c1n = _lstm_cell(gates1, c1, h)
            gates0 = x_scr[pl.ds(i * b, b), :] + _dot(h0, whh0_ref[...])
            h0n, c0n = _lstm_cell(gates0, c0, h)
            h1n = jnp.where(j > 0, h1n, h1)
            c1n = jnp.where(j > 0, c1n, c1)
            ys1_ref[pl.ds(jnp.maximum(j - 1, 0) * b, b), :] = \
                h1n.astype(ys1_ref.dtype)
            return (h0n, c0n, h1n, c1n)

        return jax.lax.fori_loop(0, _CHUNK, step, carry, unroll=8)

    zero = jnp.zeros((b, h), jnp.float32)
    h0, c0, h1, c1 = jax.lax.fori_loop(0, nchunks, chunk, (zero,) * 4)
    gates1 = _dot(h0, wih1_ref[...]) + _dot(h1, whh1_ref[...]) + bias1_ref[...]
    h1, c1 = _lstm_cell(gates1, c1, h)
    ys1_ref[pl.ds((seq - 1) * b, b), :] = h1.astype(ys1_ref.dtype)
    hn_ref[0] = h0
    cn_ref[0] = c0
    hn_ref[1] = h1
    cn_ref[1] = c1


def _head_body(ys_ref, w1_ref, b1_ref, w2_ref, b2_ref, out_ref):
    t = _dot(ys_ref[...], w1_ref[...]) + b1_ref[...]
    t = _dot(t, w2_ref[...]) + b2_ref[...]
    t = jnp.maximum(t, 0.0)
    m = jnp.max(t, axis=1, keepdims=True)
    lse = m + jnp.log(jnp.sum(jnp.exp(t - m), axis=1, keepdims=True))
    out_ref[...] = (t - lse).T


def kernel(input, embedding, W_ih_0, W_hh_0, b_ih_0, b_hh_0,
           W_ih_1, W_hh_1, b_ih_1, b_hh_1, W1, b1, W2, b2):
    s, b = input.shape
    v, e = embedding.shape
    h = W_hh_0.shape[1]
    sb = s * b

    idx = input.reshape(-1).astype(jnp.int32)
    embeds = _sc_gather(embedding, idx)

    bias0 = (b_ih_0 + b_hh_0).reshape(1, -1)
    bias1 = (b_ih_1 + b_hh_1).reshape(1, -1)

    ys1, h_n, c_n = pl.pallas_call(
        _lstm_body,
        out_shape=[
            jax.ShapeDtypeStruct((sb, h), jnp.bfloat16),
            jax.ShapeDtypeStruct((2, b, h), jnp.float32),
            jax.ShapeDtypeStruct((2, b, h), jnp.float32),
        ],
        scratch_shapes=[
            pltpu.VMEM((_CHUNK * b, 4 * h), jnp.bfloat16),
        ],
    )(embeds, W_ih_0.astype(jnp.bfloat16).T, W_hh_0.astype(jnp.bfloat16).T,
      bias0, W_ih_1.astype(jnp.bfloat16).T, W_hh_1.astype(jnp.bfloat16).T,
      bias1)

    rows = sb // _HEAD_BLOCKS
    out_t = pl.pallas_call(
        _head_body,
        grid=(_HEAD_BLOCKS,),
        in_specs=[
            pl.BlockSpec((rows, h), lambda i: (i, 0)),
            pl.BlockSpec((h, W1.shape[0]), lambda i: (0, 0)),
            pl.BlockSpec((1, W1.shape[0]), lambda i: (0, 0)),
            pl.BlockSpec((W1.shape[0], v), lambda i: (0, 0)),
            pl.BlockSpec((1, v), lambda i: (0, 0)),
        ],
        out_specs=pl.BlockSpec((v, rows), lambda i: (0, i)),
        out_shape=jax.ShapeDtypeStruct((v, sb), jnp.float32),
    )(ys1, W1.astype(jnp.bfloat16).T, b1.reshape(1, -1),
      W2.astype(jnp.bfloat16).T, b2.reshape(1, -1))

    return (out_t.T, (h_n, c_n))


# speedup vs baseline: 1.0675x; 1.0675x over previous
import jax
import jax.numpy as jnp
from jax.experimental import pallas as pl
from jax.experimental.pallas import tpu as pltpu
from jax.experimental.pallas import tpu_sc as plsc

_CHUNK = 64
_GATHER_WINDOW = 128
_HEAD_BLOCKS = 16


def _sc_gather(table, idx):
    n = idx.shape[0]
    e = table.shape[1]
    idx2 = idx.reshape(1, n)
    mesh = plsc.VectorSubcoreMesh(core_axis_name="core",
                                  subcore_axis_name="subcore")

    @pl.kernel(out_type=jax.ShapeDtypeStruct((n, e), table.dtype), mesh=mesh)
    def _gather_kernel(tab_hbm, i_hbm, o_hbm):
        def body(i_vmem, o_vmem):
            pltpu.sync_copy(tab_hbm.at[i_vmem.at[0]], o_vmem)

        pltpu.emit_pipeline(
            body,
            grid=(n // _GATHER_WINDOW,),
            in_specs=[pl.BlockSpec((1, _GATHER_WINDOW),
                                   index_map=lambda i: (0, i))],
            out_specs=[pl.BlockSpec((_GATHER_WINDOW, e),
                                    index_map=lambda i: (i, 0))],
            core_axis_name=("core", "subcore"),
            dimension_semantics=(pltpu.PARALLEL,),
        )(i_hbm, o_hbm)

    return _gather_kernel(table, idx2)


def _dot(a, b):
    return jax.lax.dot_general(a.astype(jnp.bfloat16), b.astype(jnp.bfloat16),
                               (((1,), (0,)), ((), ())),
                               preferred_element_type=jnp.float32)


def _sig(x):
    return 0.5 * jnp.tanh(0.5 * x) + 0.5


def _lstm_cell(gates, cv, h):
    ig = gates[:, :h]
    fg = gates[:, h:2 * h]
    gg = gates[:, 2 * h:3 * h]
    og = gates[:, 3 * h:]
    cc = _sig(fg) * cv + _sig(ig) * jnp.tanh(gg)
    hh = _sig(og) * jnp.tanh(cc)
    return hh, cc


def _lstm_body(embeds_ref, wih0_ref, whh0_ref, bias0_ref,
               wih1_ref, whh1_ref, bias1_ref,
               ys1_ref, hn_ref, cn_ref, x_scr, ys0_scr):
    nlayers, b, h = hn_ref.shape
    sb = embeds_ref.shape[0]
    nchunks = sb // (_CHUNK * b)

    def run_layer(src_ref, wih_ref, whh_ref, bias_ref, dst_ref, layer_idx):
        def chunk(k, hc):
            r0 = k * (_CHUNK * b)
            x_scr[...] = (_dot(src_ref[pl.ds(r0, _CHUNK * b), :],
                                wih_ref[...]) +
                          bias_ref[...]).astype(x_scr.dtype)

            def step(i, hc2):
                hv, cv = hc2
                gates = x_scr[pl.ds(i * b, b), :] + _dot(hv, whh_ref[...])
                hh, cc = _lstm_cell(gates, cv, h)
                dst_ref[pl.ds(r0 + i * b, b), :] = hh.astype(dst_ref.dtype)
                return (hh, cc)

            return jax.lax.fori_loop(0, _CHUNK, step, hc, unroll=8)

        zero = jnp.zeros((b, h), jnp.float32)
        hT, cT = jax.lax.fori_loop(0, nchunks, chunk, (zero, zero))
        hn_ref[layer_idx] = hT
        cn_ref[layer_idx] = cT

    run_layer(embeds_ref, wih0_ref, whh0_ref, bias0_ref, ys0_scr, 0)
    run_layer(ys0_scr, wih1_ref, whh1_ref, bias1_ref, ys1_ref, 1)


def _head_body(ys_ref, w1_ref, b1_ref, w2_ref, b2_ref, out_ref):
    t = _dot(ys_ref[...], w1_ref[...]) + b1_ref[...]
    t = _dot(t, w2_ref[...]) + b2_ref[...]
    t = jnp.maximum(t, 0.0)
    m = jnp.max(t, axis=1, keepdims=True)
    lse = m + jnp.log(jnp.sum(jnp.exp(t - m), axis=1, keepdims=True))
    out_ref[...] = (t - lse).T


def kernel(input, embedding, W_ih_0, W_hh_0, b_ih_0, b_hh_0,
           W_ih_1, W_hh_1, b_ih_1, b_hh_1, W1, b1, W2, b2):
    s, b = input.shape
    v, e = embedding.shape
    h = W_hh_0.shape[1]
    sb = s * b

    idx = input.reshape(-1).astype(jnp.int32)
    embeds = _sc_gather(embedding, idx)

    bias0 = (b_ih_0 + b_hh_0).reshape(1, -1)
    bias1 = (b_ih_1 + b_hh_1).reshape(1, -1)

    ys1, h_n, c_n = pl.pallas_call(
        _lstm_body,
        out_shape=[
            jax.ShapeDtypeStruct((sb, h), jnp.bfloat16),
            jax.ShapeDtypeStruct((2, b, h), jnp.float32),
            jax.ShapeDtypeStruct((2, b, h), jnp.float32),
        ],
        scratch_shapes=[
            pltpu.VMEM((_CHUNK * b, 4 * h), jnp.bfloat16),
            pltpu.VMEM((sb, h), jnp.bfloat16),
        ],
    )(embeds, W_ih_0.astype(jnp.bfloat16).T, W_hh_0.astype(jnp.bfloat16).T,
      bias0, W_ih_1.astype(jnp.bfloat16).T, W_hh_1.astype(jnp.bfloat16).T,
      bias1)

    rows = sb // _HEAD_BLOCKS
    out_t = pl.pallas_call(
        _head_body,
        grid=(_HEAD_BLOCKS,),
        in_specs=[
            pl.BlockSpec((rows, h), lambda i: (i, 0)),
            pl.BlockSpec((h, W1.shape[0]), lambda i: (0, 0)),
            pl.BlockSpec((1, W1.shape[0]), lambda i: (0, 0)),
            pl.BlockSpec((W1.shape[0], v), lambda i: (0, 0)),
            pl.BlockSpec((1, v), lambda i: (0, 0)),
        ],
        out_specs=pl.BlockSpec((v, rows), lambda i: (0, i)),
        out_shape=jax.ShapeDtypeStruct((v, sb), jnp.float32),
    )(ys1, W1.astype(jnp.bfloat16).T, b1.reshape(1, -1),
      W2.astype(jnp.bfloat16).T, b2.reshape(1, -1))

    return (out_t.T, (h_n, c_n))


# speedup vs baseline: 1.0938x; 1.0246x over previous
import jax
import jax.numpy as jnp
from jax.experimental import pallas as pl
from jax.experimental.pallas import tpu as pltpu
from jax.experimental.pallas import tpu_sc as plsc

_CHUNK = 128
_GATHER_WINDOW = 128
_HEAD_BLOCKS = 8


def _sc_gather(table, idx):
    n = idx.shape[0]
    e = table.shape[1]
    idx2 = idx.reshape(1, n)
    mesh = plsc.VectorSubcoreMesh(core_axis_name="core",
                                  subcore_axis_name="subcore")

    @pl.kernel(out_type=jax.ShapeDtypeStruct((n, e), table.dtype), mesh=mesh)
    def _gather_kernel(tab_hbm, i_hbm, o_hbm):
        def body(i_vmem, o_vmem):
            pltpu.sync_copy(tab_hbm.at[i_vmem.at[0]], o_vmem)

        pltpu.emit_pipeline(
            body,
            grid=(n // _GATHER_WINDOW,),
            in_specs=[pl.BlockSpec((1, _GATHER_WINDOW),
                                   index_map=lambda i: (0, i))],
            out_specs=[pl.BlockSpec((_GATHER_WINDOW, e),
                                    index_map=lambda i: (i, 0))],
            core_axis_name=("core", "subcore"),
            dimension_semantics=(pltpu.PARALLEL,),
        )(i_hbm, o_hbm)

    return _gather_kernel(table, idx2)


def _dot(a, b):
    return jax.lax.dot_general(a.astype(jnp.bfloat16), b.astype(jnp.bfloat16),
                               (((1,), (0,)), ((), ())),
                               preferred_element_type=jnp.float32)


def _sig(x):
    return 0.5 * jnp.tanh(0.5 * x) + 0.5


def _lstm_cell(gates, cv, h):
    ig = gates[:, :h]
    fg = gates[:, h:2 * h]
    gg = gates[:, 2 * h:3 * h]
    og = gates[:, 3 * h:]
    cc = _sig(fg) * cv + _sig(ig) * jnp.tanh(gg)
    hh = _sig(og) * jnp.tanh(cc)
    return hh, cc


def _lstm_body(embeds_ref, wih0_ref, whh0_ref, bias0_ref,
               wih1_ref, whh1_ref, bias1_ref,
               ys1_ref, hn_ref, cn_ref, x_scr, ys0_scr):
    nlayers, b, h = hn_ref.shape
    sb = embeds_ref.shape[0]
    nchunks = sb // (_CHUNK * b)

    def run_layer(src_ref, wih_ref, whh_ref, bias_ref, dst_ref, layer_idx):
        def chunk(k, hc):
            r0 = k * (_CHUNK * b)
            x_scr[...] = (_dot(src_ref[pl.ds(r0, _CHUNK * b), :],
                                wih_ref[...]) +
                          bias_ref[...]).astype(x_scr.dtype)

            def step(i, hc2):
                hv, cv = hc2
                gates = x_scr[pl.ds(i * b, b), :] + _dot(hv, whh_ref[...])
                hh, cc = _lstm_cell(gates, cv, h)
                dst_ref[pl.ds(r0 + i * b, b), :] = hh.astype(dst_ref.dtype)
                return (hh, cc)

            return jax.lax.fori_loop(0, _CHUNK, step, hc, unroll=8)

        zero = jnp.zeros((b, h), jnp.float32)
        hT, cT = jax.lax.fori_loop(0, nchunks, chunk, (zero, zero))
        hn_ref[layer_idx] = hT
        cn_ref[layer_idx] = cT

    run_layer(embeds_ref, wih0_ref, whh0_ref, bias0_ref, ys0_scr, 0)
    run_layer(ys0_scr, wih1_ref, whh1_ref, bias1_ref, ys1_ref, 1)


def _head_body(ys_ref, w1_ref, b1_ref, w2_ref, b2_ref, out_ref):
    t = _dot(ys_ref[...], w1_ref[...]) + b1_ref[...]
    t = _dot(t, w2_ref[...]) + b2_ref[...]
    t = jnp.maximum(t, 0.0)
    m = jnp.max(t, axis=1, keepdims=True)
    lse = m + jnp.log(jnp.sum(jnp.exp(t - m), axis=1, keepdims=True))
    out_ref[...] = (t - lse).T


def kernel(input, embedding, W_ih_0, W_hh_0, b_ih_0, b_hh_0,
           W_ih_1, W_hh_1, b_ih_1, b_hh_1, W1, b1, W2, b2):
    s, b = input.shape
    v, e = embedding.shape
    h = W_hh_0.shape[1]
    sb = s * b

    idx = input.reshape(-1).astype(jnp.int32)
    embeds = _sc_gather(embedding, idx)

    bias0 = (b_ih_0 + b_hh_0).reshape(1, -1)
    bias1 = (b_ih_1 + b_hh_1).reshape(1, -1)

    ys1, h_n, c_n = pl.pallas_call(
        _lstm_body,
        out_shape=[
            jax.ShapeDtypeStruct((sb, h), jnp.bfloat16),
            jax.ShapeDtypeStruct((2, b, h), jnp.float32),
            jax.ShapeDtypeStruct((2, b, h), jnp.float32),
        ],
        scratch_shapes=[
            pltpu.VMEM((_CHUNK * b, 4 * h), jnp.bfloat16),
            pltpu.VMEM((sb, h), jnp.bfloat16),
        ],
    )(embeds, W_ih_0.astype(jnp.bfloat16).T, W_hh_0.astype(jnp.bfloat16).T,
      bias0, W_ih_1.astype(jnp.bfloat16).T, W_hh_1.astype(jnp.bfloat16).T,
      bias1)

    rows = sb // _HEAD_BLOCKS
    out_t = pl.pallas_call(
        _head_body,
        grid=(_HEAD_BLOCKS,),
        in_specs=[
            pl.BlockSpec((rows, h), lambda i: (i, 0)),
            pl.BlockSpec((h, W1.shape[0]), lambda i: (0, 0)),
            pl.BlockSpec((1, W1.shape[0]), lambda i: (0, 0)),
            pl.BlockSpec((W1.shape[0], v), lambda i: (0, 0)),
            pl.BlockSpec((1, v), lambda i: (0, 0)),
        ],
        out_specs=pl.BlockSpec((v, rows), lambda i: (0, i)),
        out_shape=jax.ShapeDtypeStruct((v, sb), jnp.float32),
    )(ys1, W1.astype(jnp.bfloat16).T, b1.reshape(1, -1),
      W2.astype(jnp.bfloat16).T, b2.reshape(1, -1))

    return (out_t.T, (h_n, c_n))


# speedup vs baseline: 1.0959x; 1.0019x over previous
import jax
import jax.numpy as jnp
from jax.experimental import pallas as pl
from jax.experimental.pallas import tpu as pltpu
from jax.experimental.pallas import tpu_sc as plsc

_CHUNK = 128
_GATHER_WINDOW = 128
_HEAD_BLOCKS = 8


def _sc_gather(table, idx):
    n = idx.shape[0]
    e = table.shape[1]
    idx2 = idx.reshape(1, n)
    mesh = plsc.VectorSubcoreMesh(core_axis_name="core",
                                  subcore_axis_name="subcore")

    @pl.kernel(out_type=jax.ShapeDtypeStruct((n, e), table.dtype), mesh=mesh)
    def _gather_kernel(tab_hbm, i_hbm, o_hbm):
        def body(i_vmem, o_vmem):
            pltpu.sync_copy(tab_hbm.at[i_vmem.at[0]], o_vmem)

        pltpu.emit_pipeline(
            body,
            grid=(n // _GATHER_WINDOW,),
            in_specs=[pl.BlockSpec((1, _GATHER_WINDOW),
                                   index_map=lambda i: (0, i))],
            out_specs=[pl.BlockSpec((_GATHER_WINDOW, e),
                                    index_map=lambda i: (i, 0))],
            core_axis_name=("core", "subcore"),
            dimension_semantics=(pltpu.PARALLEL,),
        )(i_hbm, o_hbm)

    return _gather_kernel(table, idx2)


def _dot(a, b):
    return jax.lax.dot_general(a.astype(jnp.bfloat16), b.astype(jnp.bfloat16),
                               (((1,), (0,)), ((), ())),
                               preferred_element_type=jnp.float32)


def _sig(x):
    return 0.5 * jnp.tanh(0.5 * x) + 0.5


def _lstm_cell(gates, cv, h):
    ig = gates[:, :h]
    fg = gates[:, h:2 * h]
    gg = gates[:, 2 * h:3 * h]
    og = gates[:, 3 * h:]
    cc = _sig(fg) * cv + _sig(ig) * jnp.tanh(gg)
    hh = _sig(og) * jnp.tanh(cc)
    return hh, cc


def _lstm_body(embeds_ref, wih0_ref, whh0_ref, bias0_ref,
               wih1_ref, whh1_ref, bias1_ref,
               ys1_ref, hn_ref, cn_ref, x_scr, ys0_scr):
    nlayers, b, h = hn_ref.shape
    sb = embeds_ref.shape[0]
    nchunks = sb // (_CHUNK * b)

    def run_layer(src_ref, wih_ref, whh_ref, bias_ref, dst_ref, layer_idx):
        def chunk(k, hc):
            r0 = k * (_CHUNK * b)
            x_scr[...] = (_dot(src_ref[pl.ds(r0, _CHUNK * b), :],
                                wih_ref[...]) +
                          bias_ref[...]).astype(x_scr.dtype)

            def step(i, hc2):
                hv, cv = hc2
                gates = x_scr[pl.ds(i * b, b), :] + _dot(hv, whh_ref[...])
                hh, cc = _lstm_cell(gates, cv, h)
                dst_ref[pl.ds(r0 + i * b, b), :] = hh.astype(dst_ref.dtype)
                return (hh, cc)

            return jax.lax.fori_loop(0, _CHUNK, step, hc, unroll=8)

        zero = jnp.zeros((b, h), jnp.float32)
        hT, cT = jax.lax.fori_loop(0, nchunks, chunk, (zero, zero))
        hn_ref[layer_idx] = hT
        cn_ref[layer_idx] = cT

    run_layer(embeds_ref, wih0_ref, whh0_ref, bias0_ref, ys0_scr, 0)
    run_layer(ys0_scr, wih1_ref, whh1_ref, bias1_ref, ys1_ref, 1)


def _head_body(ys_ref, w1_ref, b1_ref, w2_ref, b2_ref, out_ref):
    ys_t = ys_ref[...].T
    t = _dot(w1_ref[...], ys_t) + b1_ref[...]
    t = _dot(w2_ref[...], t.astype(jnp.bfloat16)) + b2_ref[...]
    t = jnp.maximum(t, 0.0)
    m = jnp.max(t, axis=0, keepdims=True)
    lse = m + jnp.log(jnp.sum(jnp.exp(t - m), axis=0, keepdims=True))
    out_ref[...] = t - lse


def kernel(input, embedding, W_ih_0, W_hh_0, b_ih_0, b_hh_0,
           W_ih_1, W_hh_1, b_ih_1, b_hh_1, W1, b1, W2, b2):
    s, b = input.shape
    v, e = embedding.shape
    h = W_hh_0.shape[1]
    sb = s * b

    idx = input.reshape(-1).astype(jnp.int32)
    embeds = _sc_gather(embedding, idx)

    bias0 = (b_ih_0 + b_hh_0).reshape(1, -1)
    bias1 = (b_ih_1 + b_hh_1).reshape(1, -1)

    ys1, h_n, c_n = pl.pallas_call(
        _lstm_body,
        out_shape=[
            jax.ShapeDtypeStruct((sb, h), jnp.bfloat16),
            jax.ShapeDtypeStruct((2, b, h), jnp.float32),
            jax.ShapeDtypeStruct((2, b, h), jnp.float32),
        ],
        scratch_shapes=[
            pltpu.VMEM((_CHUNK * b, 4 * h), jnp.bfloat16),
            pltpu.VMEM((sb, h), jnp.bfloat16),
        ],
    )(embeds, W_ih_0.astype(jnp.bfloat16).T, W_hh_0.astype(jnp.bfloat16).T,
      bias0, W_ih_1.astype(jnp.bfloat16).T, W_hh_1.astype(jnp.bfloat16).T,
      bias1)

    rows = sb // _HEAD_BLOCKS
    out_t = pl.pallas_call(
        _head_body,
        grid=(_HEAD_BLOCKS,),
        in_specs=[
            pl.BlockSpec((rows, h), lambda i: (i, 0)),
            pl.BlockSpec((W1.shape[0], h), lambda i: (0, 0)),
            pl.BlockSpec((W1.shape[0], 1), lambda i: (0, 0)),
            pl.BlockSpec((v, W1.shape[0]), lambda i: (0, 0)),
            pl.BlockSpec((v, 1), lambda i: (0, 0)),
        ],
        out_specs=pl.BlockSpec((v, rows), lambda i: (0, i)),
        out_shape=jax.ShapeDtypeStruct((v, sb), jnp.float32),
    )(ys1, W1.astype(jnp.bfloat16), b1.reshape(-1, 1),
      W2.astype(jnp.bfloat16), b2.reshape(-1, 1))

    return (out_t.T, (h_n, c_n))


# speedup vs baseline: 1.1038x; 1.0073x over previous
import jax
import jax.numpy as jnp
from jax.experimental import pallas as pl
from jax.experimental.pallas import tpu as pltpu
from jax.experimental.pallas import tpu_sc as plsc

_CHUNK = 128
_GATHER_WINDOW = 128
_HEAD_BLOCKS = 8


def _sc_gather(table, idx):
    n = idx.shape[0]
    e = table.shape[1]
    idx2 = idx.reshape(1, n)
    mesh = plsc.VectorSubcoreMesh(core_axis_name="core",
                                  subcore_axis_name="subcore")

    @pl.kernel(out_type=jax.ShapeDtypeStruct((n, e), table.dtype), mesh=mesh)
    def _gather_kernel(tab_hbm, i_hbm, o_hbm):
        def body(i_vmem, o_vmem):
            pltpu.sync_copy(tab_hbm.at[i_vmem.at[0]], o_vmem)

        pltpu.emit_pipeline(
            body,
            grid=(n // _GATHER_WINDOW,),
            in_specs=[pl.BlockSpec((1, _GATHER_WINDOW),
                                   index_map=lambda i: (0, i))],
            out_specs=[pl.BlockSpec((_GATHER_WINDOW, e),
                                    index_map=lambda i: (i, 0))],
            core_axis_name=("core", "subcore"),
            dimension_semantics=(pltpu.PARALLEL,),
        )(i_hbm, o_hbm)

    return _gather_kernel(table, idx2)


def _dot(a, b):
    return jax.lax.dot_general(a.astype(jnp.bfloat16), b.astype(jnp.bfloat16),
                               (((1,), (0,)), ((), ())),
                               preferred_element_type=jnp.float32)


def _sig(x):
    return 0.5 * jnp.tanh(0.5 * x) + 0.5


def _lstm_cell(gates, cv, h):
    ig = gates[:, :h]
    fg = gates[:, h:2 * h]
    gg = gates[:, 2 * h:3 * h]
    og = gates[:, 3 * h:]
    cc = _sig(fg) * cv + _sig(ig) * jnp.tanh(gg)
    hh = _sig(og) * jnp.tanh(cc)
    return hh, cc


def _lstm_body(embeds_ref, wih0_ref, whh0_ref, bias0_ref,
               wih1_ref, whh1_ref, bias1_ref,
               ys1_ref, hn_ref, cn_ref, x_scr, ys0_scr):
    nlayers, b, h = hn_ref.shape
    sb = embeds_ref.shape[0]
    nchunks = sb // (_CHUNK * b)

    def run_layer(src_ref, wih_ref, whh_ref, bias_ref, dst_ref, layer_idx):
        def chunk(k, hc):
            r0 = k * (_CHUNK * b)
            x_scr[...] = (_dot(src_ref[pl.ds(r0, _CHUNK * b), :],
                                wih_ref[...]) +
                          bias_ref[...]).astype(x_scr.dtype)

            def step(i, hc2):
                hv, cv = hc2
                gates = x_scr[pl.ds(i * b, b), :] + _dot(hv, whh_ref[...])
                hh, cc = _lstm_cell(gates, cv, h)
                dst_ref[pl.ds(r0 + i * b, b), :] = hh.astype(dst_ref.dtype)
                return (hh, cc)

            return jax.lax.fori_loop(0, _CHUNK, step, hc, unroll=16)

        zero = jnp.zeros((b, h), jnp.float32)
        hT, cT = jax.lax.fori_loop(0, nchunks, chunk, (zero, zero))
        hn_ref[layer_idx] = hT
        cn_ref[layer_idx] = cT

    run_layer(embeds_ref, wih0_ref, whh0_ref, bias0_ref, ys0_scr, 0)
    run_layer(ys0_scr, wih1_ref, whh1_ref, bias1_ref, ys1_ref, 1)


def _head_body(ys_ref, w1_ref, b1_ref, w2_ref, b2_ref, out_ref):
    ys_t = ys_ref[...].T
    t = _dot(w1_ref[...], ys_t) + b1_ref[...]
    t = _dot(w2_ref[...], t.astype(jnp.bfloat16)) + b2_ref[...]
    t = jnp.maximum(t, 0.0)
    m = jnp.max(t, axis=0, keepdims=True)
    lse = m + jnp.log(jnp.sum(jnp.exp(t - m), axis=0, keepdims=True))
    out_ref[...] = t - lse


def kernel(input, embedding, W_ih_0, W_hh_0, b_ih_0, b_hh_0,
           W_ih_1, W_hh_1, b_ih_1, b_hh_1, W1, b1, W2, b2):
    s, b = input.shape
    v, e = embedding.shape
    h = W_hh_0.shape[1]
    sb = s * b

    idx = input.reshape(-1).astype(jnp.int32)
    embeds = _sc_gather(embedding, idx)

    bias0 = (b_ih_0 + b_hh_0).reshape(1, -1)
    bias1 = (b_ih_1 + b_hh_1).reshape(1, -1)

    ys1, h_n, c_n = pl.pallas_call(
        _lstm_body,
        out_shape=[
            jax.ShapeDtypeStruct((sb, h), jnp.bfloat16),
            jax.ShapeDtypeStruct((2, b, h), jnp.float32),
            jax.ShapeDtypeStruct((2, b, h), jnp.float32),
        ],
        scratch_shapes=[
            pltpu.VMEM((_CHUNK * b, 4 * h), jnp.bfloat16),
            pltpu.VMEM((sb, h), jnp.bfloat16),
        ],
    )(embeds, W_ih_0.astype(jnp.bfloat16).T, W_hh_0.astype(jnp.bfloat16).T,
      bias0, W_ih_1.astype(jnp.bfloat16).T, W_hh_1.astype(jnp.bfloat16).T,
      bias1)

    rows = sb // _HEAD_BLOCKS
    out_t = pl.pallas_call(
        _head_body,
        grid=(_HEAD_BLOCKS,),
        in_specs=[
            pl.BlockSpec((rows, h), lambda i: (i, 0)),
            pl.BlockSpec((W1.shape[0], h), lambda i: (0, 0)),
            pl.BlockSpec((W1.shape[0], 1), lambda i: (0, 0)),
            pl.BlockSpec((v, W1.shape[0]), lambda i: (0, 0)),
            pl.BlockSpec((v, 1), lambda i: (0, 0)),
        ],
        out_specs=pl.BlockSpec((v, rows), lambda i: (0, i)),
        out_shape=jax.ShapeDtypeStruct((v, sb), jnp.float32),
    )(ys1, W1.astype(jnp.bfloat16), b1.reshape(-1, 1),
      W2.astype(jnp.bfloat16), b2.reshape(-1, 1))

    return (out_t.T, (h_n, c_n))
